# Initial kernel scaffold; baseline (speedup 1.0000x reference)
import functools

import jax
import jax.numpy as jnp
from jax import lax
from jax.experimental import pallas as pl
from jax.experimental.pallas import tpu as pltpu
from jax.experimental.pallas import tpu_sc as plsc

N = 10000
D = 128
E = 320000
K = 2

NC = 2
NS = 16
NW = NC * NS
CH = 125
EW = E // NW
CPW = EW // CH
RPT = N // NS

RB = 1000
GRID = N // RB

_MESH = plsc.VectorSubcoreMesh(core_axis_name="c", subcore_axis_name="s")


@functools.partial(
    pl.kernel,
    out_type=jax.ShapeDtypeStruct((NC, N, 1), jnp.float32),
    mesh=_MESH,
    scratch_types=[
        pltpu.VMEM((CPW, CH), jnp.int32),
        pltpu.VMEM((128, 1), jnp.float32),
        pltpu.VMEM_SHARED((N, 1), jnp.float32),
    ],
)
def _deg_kernel(dst_hbm, ones_hbm, zeros_hbm, out_hbm, dstv, onesv, acc):
    c = lax.axis_index("c")
    s = lax.axis_index("s")
    w = s * NC + c
    pltpu.sync_copy(dst_hbm.at[pl.ds(w * CPW, CPW)], dstv)
    pltpu.sync_copy(ones_hbm, onesv)

    @pl.when(s == 0)
    def _():
        pltpu.sync_copy(zeros_hbm, acc)

    plsc.subcore_barrier()

    def body(i, carry):
        pltpu.sync_copy(onesv.at[pl.ds(0, CH)], acc.at[dstv.at[i]], add=True)
        return carry

    lax.fori_loop(0, CPW, body, 0)
    plsc.subcore_barrier()

    @pl.when(s == 0)
    def _():
        pltpu.sync_copy(acc, out_hbm.at[c])


@functools.partial(
    pl.kernel,
    out_type=jax.ShapeDtypeStruct((NC, N, D), jnp.float32),
    mesh=_MESH,
    scratch_types=[
        pltpu.VMEM((CPW, CH), jnp.int32),
        pltpu.VMEM((CPW, CH), jnp.int32),
        pltpu.VMEM((CH, D), jnp.float32),
        pltpu.VMEM_SHARED((N, D), jnp.float32),
        pltpu.SemaphoreType.DMA,
    ],
)
def _prop_kernel(u_hbm, src_hbm, dst_hbm, zeros_hbm, out_hbm,
                 srcv, dstv, buf, acc, gsem):
    c = lax.axis_index("c")
    s = lax.axis_index("s")
    w = s * NC + c
    pltpu.sync_copy(src_hbm.at[pl.ds(w * CPW, CPW)], srcv)
    pltpu.sync_copy(dst_hbm.at[pl.ds(w * CPW, CPW)], dstv)
    pltpu.sync_copy(zeros_hbm.at[pl.ds(s * RPT, RPT)],
                    acc.at[pl.ds(s * RPT, RPT)])
    plsc.subcore_barrier()

    def body(i, carry):
        pltpu.async_copy(u_hbm.at[srcv.at[i]], buf, gsem).wait()
        pltpu.sync_copy(buf, acc.at[dstv.at[i]], add=True)
        return carry

    lax.fori_loop(0, CPW, body, 0)
    plsc.subcore_barrier()
    pltpu.sync_copy(acc.at[pl.ds(s * RPT, RPT)],
                    out_hbm.at[c].at[pl.ds(s * RPT, RPT)])


def _norm_u0_body(dega_ref, degb_ref, feat_ref, norm_ref, u0_ref):
    i = pl.program_id(0)
    deg = dega_ref[...] + degb_ref[...]
    nrm = lax.rsqrt(jnp.maximum(deg, 1.0))
    norm_ref[...] = nrm
    nb = lax.dynamic_slice(nrm, (i * RB, 0), (RB, 1))
    u0_ref[...] = feat_ref[...] * nb


_norm_u0 = pl.pallas_call(
    _norm_u0_body,
    grid=(GRID,),
    in_specs=[
        pl.BlockSpec((N, 1), lambda i: (0, 0)),
        pl.BlockSpec((N, 1), lambda i: (0, 0)),
        pl.BlockSpec((RB, D), lambda i: (i, 0)),
    ],
    out_specs=[
        pl.BlockSpec((N, 1), lambda i: (0, 0)),
        pl.BlockSpec((RB, D), lambda i: (i, 0)),
    ],
    out_shape=[
        jax.ShapeDtypeStruct((N, 1), jnp.float32),
        jax.ShapeDtypeStruct((N, D), jnp.float32),
    ],
)


def _combine_body(pa_ref, pb_ref, norm_ref, f_ref, u_ref):
    i = pl.program_id(0)
    nb = lax.dynamic_slice(norm_ref[...], (i * RB, 0), (RB, 1))
    f = (pa_ref[...] + pb_ref[...]) * nb
    f_ref[...] = f
    u_ref[...] = f * nb


_combine = pl.pallas_call(
    _combine_body,
    grid=(GRID,),
    in_specs=[
        pl.BlockSpec((RB, D), lambda i: (i, 0)),
        pl.BlockSpec((RB, D), lambda i: (i, 0)),
        pl.BlockSpec((N, 1), lambda i: (0, 0)),
    ],
    out_specs=[
        pl.BlockSpec((RB, D), lambda i: (i, 0)),
        pl.BlockSpec((RB, D), lambda i: (i, 0)),
    ],
    out_shape=[
        jax.ShapeDtypeStruct((N, D), jnp.float32),
        jax.ShapeDtypeStruct((N, D), jnp.float32),
    ],
)


def _mm1_body(f0_ref, f1_ref, pa_ref, pb_ref, norm_ref, w_ref, b_ref,
              h_ref, u_ref):
    i = pl.program_id(0)
    nb = lax.dynamic_slice(norm_ref[...], (i * RB, 0), (RB, 1))
    f2 = (pa_ref[...] + pb_ref[...]) * nb
    w = w_ref[...]
    h = jnp.dot(f0_ref[...], w[0:D], preferred_element_type=jnp.float32)
    h = h + jnp.dot(f1_ref[...], w[D:2 * D], preferred_element_type=jnp.float32)
    h = h + jnp.dot(f2, w[2 * D:3 * D], preferred_element_type=jnp.float32)
    h = h + b_ref[...]
    h_ref[...] = h
    u_ref[...] = h * nb


_mm1 = pl.pallas_call(
    _mm1_body,
    grid=(GRID,),
    in_specs=[
        pl.BlockSpec((RB, D), lambda i: (i, 0)),
        pl.BlockSpec((RB, D), lambda i: (i, 0)),
        pl.BlockSpec((RB, D), lambda i: (i, 0)),
        pl.BlockSpec((RB, D), lambda i: (i, 0)),
        pl.BlockSpec((N, 1), lambda i: (0, 0)),
        pl.BlockSpec(((K + 1) * D, D), lambda i: (0, 0)),
        pl.BlockSpec((1, D), lambda i: (0, 0)),
    ],
    out_specs=[
        pl.BlockSpec((RB, D), lambda i: (i, 0)),
        pl.BlockSpec((RB, D), lambda i: (i, 0)),
    ],
    out_shape=[
        jax.ShapeDtypeStruct((N, D), jnp.float32),
        jax.ShapeDtypeStruct((N, D), jnp.float32),
    ],
)


def _mm2_body(h1_ref, f1_ref, pa_ref, pb_ref, norm_ref, w_ref, b_ref,
              wfc_ref, bfc_ref, out_ref):
    i = pl.program_id(0)
    nb = lax.dynamic_slice(norm_ref[...], (i * RB, 0), (RB, 1))
    f2 = (pa_ref[...] + pb_ref[...]) * nb
    w = w_ref[...]
    h = jnp.dot(h1_ref[...], w[0:D], preferred_element_type=jnp.float32)
    h = h + jnp.dot(f1_ref[...], w[D:2 * D], preferred_element_type=jnp.float32)
    h = h + jnp.dot(f2, w[2 * D:3 * D], preferred_element_type=jnp.float32)
    h = h + b_ref[...]
    out_ref[...] = jnp.dot(h, wfc_ref[...],
                           preferred_element_type=jnp.float32) + bfc_ref[...]


_mm2 = pl.pallas_call(
    _mm2_body,
    grid=(GRID,),
    in_specs=[
        pl.BlockSpec((RB, D), lambda i: (i, 0)),
        pl.BlockSpec((RB, D), lambda i: (i, 0)),
        pl.BlockSpec((RB, D), lambda i: (i, 0)),
        pl.BlockSpec((RB, D), lambda i: (i, 0)),
        pl.BlockSpec((N, 1), lambda i: (0, 0)),
        pl.BlockSpec(((K + 1) * D, D), lambda i: (0, 0)),
        pl.BlockSpec((1, D), lambda i: (0, 0)),
        pl.BlockSpec((D, D), lambda i: (0, 0)),
        pl.BlockSpec((1, D), lambda i: (0, 0)),
    ],
    out_specs=pl.BlockSpec((RB, D), lambda i: (i, 0)),
    out_shape=jax.ShapeDtypeStruct((N, D), jnp.float32),
)


def kernel(features, edge_index, W1, b1, W2, b2, Wfc, bfc):
    src = edge_index[0].reshape(E // CH, CH)
    dst = edge_index[1].reshape(E // CH, CH)
    zeros_nd = jnp.zeros((N, D), jnp.float32)
    zeros_n1 = jnp.zeros((N, 1), jnp.float32)
    ones = jnp.ones((128, 1), jnp.float32)

    degp = _deg_kernel(dst, ones, zeros_n1)
    norm, u0 = _norm_u0(degp[0], degp[1], features)

    p1 = _prop_kernel(u0, src, dst, zeros_nd)
    f1, u1 = _combine(p1[0], p1[1], norm)
    p2 = _prop_kernel(u1, src, dst, zeros_nd)
    h1, u0b = _mm1(features, f1, p2[0], p2[1], norm, W1, b1.reshape(1, D))

    p1b = _prop_kernel(u0b, src, dst, zeros_nd)
    f1b, u1b = _combine(p1b[0], p1b[1], norm)
    p2b = _prop_kernel(u1b, src, dst, zeros_nd)
    out = _mm2(h1, f1b, p2b[0], p2b[1], norm, W2, b2.reshape(1, D),
               Wfc, bfc.reshape(1, D))
    return out


# speedup vs baseline: 6.9979x; 6.9979x over previous
import functools

import jax
import jax.numpy as jnp
from jax import lax
from jax.experimental import pallas as pl
from jax.experimental.pallas import tpu as pltpu
from jax.experimental.pallas import tpu_sc as plsc

N = 10000
D = 128
E = 320000
K = 2

NC = 2
NS = 16
NW = NC * NS
CH = 125
EW = E // NW
CPW = EW // CH
NP = 10240
RPT = NP // NS

RB = 1000
GRID = N // RB

_MESH = plsc.VectorSubcoreMesh(core_axis_name="c", subcore_axis_name="s")


@functools.partial(
    pl.kernel,
    out_type=jax.ShapeDtypeStruct((NC, N, 1), jnp.float32),
    mesh=_MESH,
    scratch_types=[
        pltpu.VMEM((CPW, CH), jnp.int32),
        pltpu.VMEM((128, 1), jnp.float32),
        pltpu.VMEM_SHARED((N, 1), jnp.float32),
    ],
)
def _deg_kernel(dst_hbm, ones_hbm, zeros_hbm, out_hbm, dstv, onesv, acc):
    c = lax.axis_index("c")
    s = lax.axis_index("s")
    w = s * NC + c
    pltpu.sync_copy(dst_hbm.at[pl.ds(w * CPW, CPW)], dstv)
    pltpu.sync_copy(ones_hbm, onesv)

    @pl.when(s == 0)
    def _():
        pltpu.sync_copy(zeros_hbm, acc)

    plsc.subcore_barrier()

    def body(i, carry):
        pltpu.sync_copy(onesv.at[pl.ds(0, CH)], acc.at[dstv.at[i]], add=True)
        return carry

    lax.fori_loop(0, CPW, body, 0)
    plsc.subcore_barrier()

    @pl.when(s == 0)
    def _():
        pltpu.sync_copy(acc, out_hbm.at[c])


@functools.partial(
    pl.kernel,
    out_type=jax.ShapeDtypeStruct((NC, NP, D), jnp.float32),
    mesh=_MESH,
    scratch_types=[
        pltpu.VMEM((CPW, CH), jnp.int32),
        pltpu.VMEM((CPW, CH), jnp.int32),
        pltpu.VMEM((CH, D), jnp.float32),
        pltpu.VMEM_SHARED((NP, D), jnp.float32),
        pltpu.SemaphoreType.DMA,
    ],
)
def _prop_kernel(u_hbm, src_hbm, dst_hbm, zeros_hbm, out_hbm,
                 srcv, dstv, buf, acc, gsem):
    c = lax.axis_index("c")
    s = lax.axis_index("s")
    w = s * NC + c
    pltpu.sync_copy(src_hbm.at[pl.ds(w * CPW, CPW)], srcv)
    pltpu.sync_copy(dst_hbm.at[pl.ds(w * CPW, CPW)], dstv)
    pltpu.sync_copy(zeros_hbm.at[pl.ds(s * RPT, RPT)],
                    acc.at[pl.ds(s * RPT, RPT)])
    plsc.subcore_barrier()

    def body(i, carry):
        pltpu.async_copy(u_hbm.at[srcv.at[i]], buf, gsem).wait()
        pltpu.sync_copy(buf, acc.at[dstv.at[i]], add=True)
        return carry

    lax.fori_loop(0, CPW, body, 0)
    plsc.subcore_barrier()
    pltpu.sync_copy(acc.at[pl.ds(s * RPT, RPT)],
                    out_hbm.at[c].at[pl.ds(s * RPT, RPT)])


def _norm_u0_body(dega_ref, degb_ref, feat_ref, norm_ref, u0_ref):
    i = pl.program_id(0)
    deg = dega_ref[...] + degb_ref[...]
    nrm = lax.rsqrt(jnp.maximum(deg, 1.0))
    norm_ref[...] = nrm
    nb = norm_ref[pl.ds(i * RB, RB), :]
    u0_ref[...] = feat_ref[...] * nb


_norm_u0 = pl.pallas_call(
    _norm_u0_body,
    grid=(GRID,),
    in_specs=[
        pl.BlockSpec((N, 1), lambda i: (0, 0)),
        pl.BlockSpec((N, 1), lambda i: (0, 0)),
        pl.BlockSpec((RB, D), lambda i: (i, 0)),
    ],
    out_specs=[
        pl.BlockSpec((N, 1), lambda i: (0, 0)),
        pl.BlockSpec((RB, D), lambda i: (i, 0)),
    ],
    out_shape=[
        jax.ShapeDtypeStruct((N, 1), jnp.float32),
        jax.ShapeDtypeStruct((N, D), jnp.float32),
    ],
)


def _combine_body(pa_ref, pb_ref, norm_ref, f_ref, u_ref):
    i = pl.program_id(0)
    nb = norm_ref[pl.ds(i * RB, RB), :]
    f = (pa_ref[...] + pb_ref[...]) * nb
    f_ref[...] = f
    u_ref[...] = f * nb


_combine = pl.pallas_call(
    _combine_body,
    grid=(GRID,),
    in_specs=[
        pl.BlockSpec((RB, D), lambda i: (i, 0)),
        pl.BlockSpec((RB, D), lambda i: (i, 0)),
        pl.BlockSpec((N, 1), lambda i: (0, 0)),
    ],
    out_specs=[
        pl.BlockSpec((RB, D), lambda i: (i, 0)),
        pl.BlockSpec((RB, D), lambda i: (i, 0)),
    ],
    out_shape=[
        jax.ShapeDtypeStruct((N, D), jnp.float32),
        jax.ShapeDtypeStruct((N, D), jnp.float32),
    ],
)


def _mm1_body(f0_ref, f1_ref, pa_ref, pb_ref, norm_ref, w_ref, b_ref,
              h_ref, u_ref):
    i = pl.program_id(0)
    nb = norm_ref[pl.ds(i * RB, RB), :]
    f2 = (pa_ref[...] + pb_ref[...]) * nb
    w = w_ref[...]
    h = jnp.dot(f0_ref[...], w[0:D], preferred_element_type=jnp.float32)
    h = h + jnp.dot(f1_ref[...], w[D:2 * D], preferred_element_type=jnp.float32)
    h = h + jnp.dot(f2, w[2 * D:3 * D], preferred_element_type=jnp.float32)
    h = h + b_ref[...]
    h_ref[...] = h
    u_ref[...] = h * nb


_mm1 = pl.pallas_call(
    _mm1_body,
    grid=(GRID,),
    in_specs=[
        pl.BlockSpec((RB, D), lambda i: (i, 0)),
        pl.BlockSpec((RB, D), lambda i: (i, 0)),
        pl.BlockSpec((RB, D), lambda i: (i, 0)),
        pl.BlockSpec((RB, D), lambda i: (i, 0)),
        pl.BlockSpec((N, 1), lambda i: (0, 0)),
        pl.BlockSpec(((K + 1) * D, D), lambda i: (0, 0)),
        pl.BlockSpec((1, D), lambda i: (0, 0)),
    ],
    out_specs=[
        pl.BlockSpec((RB, D), lambda i: (i, 0)),
        pl.BlockSpec((RB, D), lambda i: (i, 0)),
    ],
    out_shape=[
        jax.ShapeDtypeStruct((N, D), jnp.float32),
        jax.ShapeDtypeStruct((N, D), jnp.float32),
    ],
)


def _mm2_body(h1_ref, f1_ref, pa_ref, pb_ref, norm_ref, w_ref, b_ref,
              wfc_ref, bfc_ref, out_ref):
    i = pl.program_id(0)
    nb = norm_ref[pl.ds(i * RB, RB), :]
    f2 = (pa_ref[...] + pb_ref[...]) * nb
    w = w_ref[...]
    h = jnp.dot(h1_ref[...], w[0:D], preferred_element_type=jnp.float32)
    h = h + jnp.dot(f1_ref[...], w[D:2 * D], preferred_element_type=jnp.float32)
    h = h + jnp.dot(f2, w[2 * D:3 * D], preferred_element_type=jnp.float32)
    h = h + b_ref[...]
    out_ref[...] = jnp.dot(h, wfc_ref[...],
                           preferred_element_type=jnp.float32) + bfc_ref[...]


_mm2 = pl.pallas_call(
    _mm2_body,
    grid=(GRID,),
    in_specs=[
        pl.BlockSpec((RB, D), lambda i: (i, 0)),
        pl.BlockSpec((RB, D), lambda i: (i, 0)),
        pl.BlockSpec((RB, D), lambda i: (i, 0)),
        pl.BlockSpec((RB, D), lambda i: (i, 0)),
        pl.BlockSpec((N, 1), lambda i: (0, 0)),
        pl.BlockSpec(((K + 1) * D, D), lambda i: (0, 0)),
        pl.BlockSpec((1, D), lambda i: (0, 0)),
        pl.BlockSpec((D, D), lambda i: (0, 0)),
        pl.BlockSpec((1, D), lambda i: (0, 0)),
    ],
    out_specs=pl.BlockSpec((RB, D), lambda i: (i, 0)),
    out_shape=jax.ShapeDtypeStruct((N, D), jnp.float32),
)


def kernel(features, edge_index, W1, b1, W2, b2, Wfc, bfc):
    src = edge_index[0].reshape(E // CH, CH)
    dst = edge_index[1].reshape(E // CH, CH)
    zeros_nd = jnp.zeros((NP, D), jnp.float32)
    zeros_n1 = jnp.zeros((N, 1), jnp.float32)
    ones = jnp.ones((128, 1), jnp.float32)

    degp = _deg_kernel(dst, ones, zeros_n1)
    norm, u0 = _norm_u0(degp[0], degp[1], features)

    p1 = _prop_kernel(u0, src, dst, zeros_nd)
    f1, u1 = _combine(p1[0], p1[1], norm)
    p2 = _prop_kernel(u1, src, dst, zeros_nd)
    h1, u0b = _mm1(features, f1, p2[0], p2[1], norm, W1, b1.reshape(1, D))

    p1b = _prop_kernel(u0b, src, dst, zeros_nd)
    f1b, u1b = _combine(p1b[0], p1b[1], norm)
    p2b = _prop_kernel(u1b, src, dst, zeros_nd)
    out = _mm2(h1, f1b, p2b[0], p2b[1], norm, W2, b2.reshape(1, D),
               Wfc, bfc.reshape(1, D))
    return out
